# baseline (speedup 1.0000x reference)
import functools

import jax
import jax.numpy as jnp
from jax import lax
from jax.experimental import pallas as pl
from jax.experimental.pallas import tpu as pltpu
from jax.experimental.pallas import tpu_sc as plsc


def _chunk_plan(per_w: int, main: int):
    taper = [48, 56, 96, 200]
    if per_w >= 2 * sum(taper) + main and (per_w - 2 * sum(taper)) % main == 0:
        n_main = (per_w - 2 * sum(taper)) // main
        return taper + [main] * n_main + taper[::-1]
    chunk = main
    while per_w % chunk != 0:
        chunk //= 2
    return [chunk] * (per_w // chunk)


def _make_gather(total: int, vocab: int, dim: int):
    info = plsc.get_sparse_core_info()
    nc, ns = info.num_cores, info.num_subcores
    nw = nc * ns
    assert total % nw == 0
    per_w = total // nw
    chunks = _chunk_plan(per_w, 400)
    starts = [0]
    for c in chunks:
        starts.append(starts[-1] + c)
    n_chunks = len(chunks)
    bufrows = max(chunks)
    nbuf = 4
    idx_split = min(400, per_w)

    mesh = plsc.VectorSubcoreMesh(core_axis_name="c", subcore_axis_name="s")

    @functools.partial(
        pl.kernel,
        out_type=jax.ShapeDtypeStruct((total, dim), jnp.float32),
        mesh=mesh,
        scratch_types=[
            pltpu.VMEM((per_w,), jnp.int32),
            [pltpu.VMEM((bufrows, dim), jnp.float32) for _ in range(nbuf)],
            [pltpu.SemaphoreType.DMA for _ in range(nbuf)],
            [pltpu.SemaphoreType.DMA for _ in range(nbuf)],
        ],
        compiler_params=pltpu.CompilerParams(use_tc_tiling_on_sc=False),
    )
    def gather(table_hbm, idx_hbm, out_hbm, idx_v, rows, gsems, wsems):
        wid = lax.axis_index("s") * nc + lax.axis_index("c")
        base = wid * per_w
        pltpu.sync_copy(idx_hbm.at[pl.ds(base, idx_split)],
                        idx_v.at[pl.ds(0, idx_split)])

        def issue_gather(c, b):
            pltpu.async_copy(
                table_hbm.at[idx_v.at[pl.ds(starts[c], chunks[c])]],
                rows[b].at[pl.ds(0, chunks[c])], gsems[b],
            )

        def wait_gather(c, b):
            pltpu.make_async_copy(
                table_hbm.at[idx_v.at[pl.ds(starts[c], chunks[c])]],
                rows[b].at[pl.ds(0, chunks[c])], gsems[b],
            ).wait()

        def issue_write(c, b):
            pltpu.async_copy(
                rows[b].at[pl.ds(0, chunks[c])],
                out_hbm.at[pl.ds(base + starts[c], chunks[c])], wsems[b],
            )

        def wait_write(c, b):
            pltpu.make_async_copy(
                rows[b].at[pl.ds(0, chunks[c])],
                out_hbm.at[pl.ds(base + starts[c], chunks[c])], wsems[b],
            ).wait()

        n_first = 0
        while starts[n_first + 1] <= idx_split and n_first + 1 < n_chunks:
            n_first += 1
        n_first = min(n_first, nbuf)
        for c in range(n_first):
            issue_gather(c, c % nbuf)
        if idx_split < per_w:
            pltpu.sync_copy(idx_hbm.at[pl.ds(base + idx_split, per_w - idx_split)],
                            idx_v.at[pl.ds(idx_split, per_w - idx_split)])

        for i in range(n_chunks + nbuf - 1):
            if n_first <= i < n_chunks:
                b = i % nbuf
                if i >= nbuf:
                    wait_write(i - nbuf, b)
                issue_gather(i, b)
            j = i - (nbuf - 1)
            if 0 <= j < n_chunks:
                b = j % nbuf
                wait_gather(j, b)
                issue_write(j, b)
        for j in range(max(0, n_chunks - nbuf), n_chunks):
            wait_write(j, j % nbuf)

    return gather


def kernel(indices, table):
    b, l = indices.shape
    vocab, dim = table.shape
    flat = indices.reshape(b * l)
    gather = _make_gather(b * l, vocab, dim)
    out = gather(table, flat)
    return out.reshape(b, l, dim)
